# baseline (speedup 1.0000x reference)
import functools

import jax
import jax.numpy as jnp
from jax import lax
from jax.experimental import pallas as pl
from jax.experimental.pallas import tpu as pltpu
from jax.experimental.pallas import tpu_sc as plsc

_BATCH = 16384
_WIDTH = 64


def _build_gather():
    info = plsc.get_sparse_core_info()
    nc, ns, nl = info.num_cores, info.num_subcores, info.num_lanes
    nw = nc * ns
    bpw = _BATCH // nw
    nch = 2
    chunk = bpw // nch
    ng = chunk // nl

    mesh = plsc.VectorSubcoreMesh(core_axis_name="c", subcore_axis_name="s")

    @functools.partial(
        pl.kernel,
        mesh=mesh,
        out_type=jax.ShapeDtypeStruct((_BATCH, _WIDTH), jnp.float32),
        scratch_types=[
            pltpu.VMEM((chunk,), jnp.int32),
            pltpu.VMEM((chunk,), jnp.int32),
            pltpu.VMEM((chunk, 2 * _WIDTH), jnp.float32),
            pltpu.VMEM((chunk, _WIDTH), jnp.float32),
            pltpu.SemaphoreType.DMA,
        ],
        compiler_params=pltpu.CompilerParams(needs_layout_passes=False),
    )
    def gather_kernel(idx_hbm, table_hbm, out_hbm, idx_v, row_v, rows_v, out_v, sem):
        wid = lax.axis_index("s") * nc + lax.axis_index("c")

        for ch in range(nch):
            base = wid * bpw + ch * chunk
            pltpu.sync_copy(idx_hbm.at[pl.ds(base, chunk)], idx_v)

            def prep(g, carry):
                v = idx_v[pl.ds(g * nl, nl)]
                row_v[pl.ds(g * nl, nl)] = lax.shift_right_logical(v, 1)
                return carry

            lax.fori_loop(0, ng, prep, 0)

            pltpu.async_copy(table_hbm.at[row_v], rows_v, sem).wait()

            def select(g, carry):
                kvec = lax.iota(jnp.int32, nl) + g * nl
                half = (idx_v[pl.ds(g * nl, nl)] & 1) * _WIDTH
                for c in range(_WIDTH):
                    vals = plsc.load_gather(rows_v, [kvec, half + c])
                    plsc.store_scatter(
                        out_v, [kvec, jnp.full((nl,), c, jnp.int32)], vals)
                return carry

            lax.fori_loop(0, ng, select, 0)

            pltpu.sync_copy(out_v, out_hbm.at[pl.ds(base, chunk)])

    return gather_kernel


_gather = _build_gather()


def kernel(y, genre_emb):
    table2 = genre_emb.reshape(genre_emb.shape[0] // 2, 2 * _WIDTH)
    out = _gather(y.astype(jnp.int32), table2)
    return out[:, None, :]


# speedup vs baseline: 1.7982x; 1.7982x over previous
import functools

import jax
import jax.numpy as jnp
from jax import lax
from jax.experimental import pallas as pl
from jax.experimental.pallas import tpu as pltpu
from jax.experimental.pallas import tpu_sc as plsc

_BATCH = 16384
_WIDTH = 64


def _build_gather():
    info = plsc.get_sparse_core_info()
    nc, ns, nl = info.num_cores, info.num_subcores, info.num_lanes
    nw = nc * ns
    bpw = _BATCH // nw
    nch = 4
    chunk = bpw // nch

    mesh = plsc.VectorSubcoreMesh(core_axis_name="c", subcore_axis_name="s")

    @functools.partial(
        pl.kernel,
        mesh=mesh,
        out_type=jax.ShapeDtypeStruct((_BATCH, _WIDTH), jnp.float32),
        scratch_types=[
            pltpu.VMEM((chunk,), jnp.int32),
            pltpu.VMEM((chunk, _WIDTH), jnp.float32),
            pltpu.SemaphoreType.DMA,
        ],
        compiler_params=pltpu.CompilerParams(needs_layout_passes=False),
    )
    def gather_kernel(idx_hbm, table_hbm, out_hbm, idx_v, rows_v, sem):
        wid = lax.axis_index("s") * nc + lax.axis_index("c")

        for ch in range(nch):
            base = wid * bpw + ch * chunk
            pltpu.sync_copy(idx_hbm.at[pl.ds(base, chunk)], idx_v)

            def fire(g, carry):
                v = idx_v[pl.ds(g * nl, nl)]
                for j in range(nl):
                    pltpu.async_copy(
                        table_hbm.at[v[j]], rows_v.at[g * nl + j], sem)
                return carry

            lax.fori_loop(0, chunk // nl, fire, 0)
            pltpu.make_async_copy(
                table_hbm.at[pl.ds(0, chunk)], rows_v, sem).wait()

            pltpu.sync_copy(rows_v, out_hbm.at[pl.ds(base, chunk)])

    return gather_kernel


_gather = _build_gather()


def kernel(y, genre_emb):
    out = _gather(y.astype(jnp.int32), genre_emb)
    return out[:, None, :]
